# baseline (speedup 1.0000x reference)
import jax
import jax.numpy as jnp
from jax import lax
from jax.experimental import pallas as pl
from jax.experimental.pallas import tpu as pltpu
from jax.experimental.pallas import tpu_sc as plsc

_NC, _NS = 2, 16
_NW = _NC * _NS


def _add_body(x_ref, pos_ref, out_ref):
    out_ref[...] = x_ref[...] + pos_ref[...]


def _make_sc_body(batch, seq_len, d_model, sc_rows):
    start = (seq_len - sc_rows) * d_model
    ch = sc_rows * d_model // _NW

    def body(xf_hbm, pf_hbm, out_hbm, xbuf, posbuf):
        cid = lax.axis_index("c")
        sid = lax.axis_index("s")
        wid = sid * _NC + cid
        off = wid * ch
        pltpu.sync_copy(pf_hbm.at[pl.ds(start + off, ch)], posbuf)
        for b in range(batch):
            pltpu.sync_copy(xf_hbm.at[b, pl.ds(start + off, ch)], xbuf)

            @plsc.parallel_loop(0, ch, step=16, unroll=8)
            def _(i):
                xbuf[pl.ds(i, 16)] = xbuf[pl.ds(i, 16)] + posbuf[pl.ds(i, 16)]

            pltpu.sync_copy(xbuf, out_hbm.at[b, pl.ds(off, ch)])

    return body, ch


def kernel(x, pos_table):
    batch, seq_len, d_model = x.shape
    sc_rows = seq_len // 8
    tc_rows = seq_len - sc_rows
    blk_s = 896
    tc_out = pl.pallas_call(
        _add_body,
        grid=(tc_rows // blk_s, batch),
        in_specs=[
            pl.BlockSpec((1, blk_s, d_model), lambda s, b: (b, s, 0)),
            pl.BlockSpec((blk_s, d_model), lambda s, b: (s, 0)),
        ],
        out_specs=pl.BlockSpec((1, blk_s, d_model), lambda s, b: (b, s, 0)),
        out_shape=jax.ShapeDtypeStruct((batch, tc_rows, d_model), x.dtype),
    )(x, pos_table)

    sc_body, ch = _make_sc_body(batch, seq_len, d_model, sc_rows)
    sc_out = pl.kernel(
        sc_body,
        out_type=jax.ShapeDtypeStruct((batch, sc_rows * d_model), x.dtype),
        mesh=plsc.VectorSubcoreMesh(core_axis_name="c", subcore_axis_name="s"),
        scratch_types=[
            pltpu.VMEM((ch,), jnp.float32),
            pltpu.VMEM((ch,), jnp.float32),
        ],
    )(x.reshape(batch, seq_len * d_model), pos_table.reshape(-1))

    return jnp.concatenate(
        [tc_out, sc_out.reshape(batch, sc_rows, d_model)], axis=1
    )


# speedup vs baseline: 4.4039x; 4.4039x over previous
import jax
import jax.numpy as jnp
from jax.experimental import pallas as pl


def _add_kernel(x_ref, pos_ref, out_ref):
    out_ref[...] = x_ref[...] + pos_ref[...]


def kernel(x, pos_table):
    batch, seq_len, d_model = x.shape
    blk_s = 2048
    grid = (seq_len // blk_s, batch)
    return pl.pallas_call(
        _add_kernel,
        grid=grid,
        in_specs=[
            pl.BlockSpec((1, blk_s, d_model), lambda s, b: (b, s, 0)),
            pl.BlockSpec((blk_s, d_model), lambda s, b: (s, 0)),
        ],
        out_specs=pl.BlockSpec((1, blk_s, d_model), lambda s, b: (b, s, 0)),
        out_shape=jax.ShapeDtypeStruct(x.shape, x.dtype),
    )(x, pos_table)
